# baseline (speedup 1.0000x reference)
import jax
import jax.numpy as jnp
from jax.experimental import pallas as pl
from jax.experimental.pallas import tpu as pltpu

N = 2048
BM = BN = BK = 512
KB = N // BK
BI = 512
BR = 128


def _build_dense(edge_indices, edge_values):
    def one(ei, ev):
        return jnp.zeros((N, N), dtype=ev.dtype).at[ei[0], ei[1]].add(ev)
    return jax.vmap(one)(edge_indices, edge_values)


def _combo_body(f_ref, a_ref, out_ref):
    for c in range(6):
        acc = f_ref[c, 0] * a_ref[0]
        for e in range(1, 5):
            acc = acc + f_ref[c, e] * a_ref[e]
        out_ref[c] = acc


def _combos(F, A):
    return pl.pallas_call(
        _combo_body,
        grid=(N // BR,),
        in_specs=[
            pl.BlockSpec(memory_space=pltpu.SMEM),
            pl.BlockSpec((5, BR, N), lambda i: (0, i, 0)),
        ],
        out_specs=pl.BlockSpec((6, BR, N), lambda i: (0, i, 0)),
        out_shape=jax.ShapeDtypeStruct((6, N, N), jnp.float32),
        compiler_params=pltpu.CompilerParams(
            dimension_semantics=("arbitrary",)),
    )(F, A)


def _mm_plain_body(a_ref, b_ref, out_ref, acc_ref):
    i = pl.program_id(1)
    j = pl.program_id(2)
    k = pl.program_id(3)

    @pl.when(k == 0)
    def _():
        acc_ref[...] = jnp.zeros_like(acc_ref)

    acc_ref[...] += jnp.dot(a_ref[0], b_ref[0],
                            preferred_element_type=jnp.float32)

    @pl.when(k == KB - 1)
    def _():
        r = acc_ref[...]
        ir = jax.lax.broadcasted_iota(jnp.int32, (BM, BN), 0) + i * BM
        ic = jax.lax.broadcasted_iota(jnp.int32, (BM, BN), 1) + j * BN
        out_ref[0] = jnp.where(ir == ic, 0.0, r)


def _mm_scaled_body(cs_ref, a_ref, b_ref, out_ref, acc_ref):
    i = pl.program_id(1)
    j = pl.program_id(2)
    k = pl.program_id(3)

    @pl.when(k == 0)
    def _():
        acc_ref[...] = jnp.zeros_like(acc_ref)

    sc = cs_ref[0, 0]
    dinv = jnp.where(sc != 0, 1.0 / jnp.where(sc != 0, sc, 1.0), 0.0)
    acc_ref[...] += jnp.dot(a_ref[0] * dinv, b_ref[0],
                            preferred_element_type=jnp.float32)

    @pl.when(k == KB - 1)
    def _():
        r = acc_ref[...]
        ir = jax.lax.broadcasted_iota(jnp.int32, (BM, BN), 0) + i * BM
        ic = jax.lax.broadcasted_iota(jnp.int32, (BM, BN), 1) + j * BN
        out_ref[0] = jnp.where(ir == ic, 0.0, r)


def _mm(a, b, cs=None):
    grid = (2, N // BM, N // BN, KB)
    specs = [
        pl.BlockSpec((1, BM, BK), lambda c, i, j, k: (c, i, k)),
        pl.BlockSpec((1, BK, BN), lambda c, i, j, k: (c, k, j)),
    ]
    args = [a, b]
    body = _mm_plain_body
    if cs is not None:
        specs.insert(0, pl.BlockSpec((1, 1, 1, BK),
                                     lambda c, i, j, k: (c, k, 0, 0)))
        args.insert(0, cs.reshape(2, KB, 1, BK))
        body = _mm_scaled_body
    return pl.pallas_call(
        body,
        grid=grid,
        in_specs=specs,
        out_specs=pl.BlockSpec((1, BM, BN), lambda c, i, j, k: (c, i, j)),
        out_shape=jax.ShapeDtypeStruct((2, N, N), jnp.float32),
        scratch_shapes=[pltpu.VMEM((BM, BN), jnp.float32)],
        compiler_params=pltpu.CompilerParams(
            dimension_semantics=("parallel", "parallel", "arbitrary",
                                 "arbitrary")),
    )(*args)


def _colsum_body(h_ref, out_ref):
    i = pl.program_id(1)

    @pl.when(i == 0)
    def _():
        out_ref[...] = jnp.zeros_like(out_ref)

    out_ref[0] += jnp.sum(h_ref[0], axis=0, keepdims=True)


def _colsum(h):
    blk = 256
    return pl.pallas_call(
        _colsum_body,
        grid=(2, N // blk),
        in_specs=[pl.BlockSpec((1, blk, N), lambda c, i: (c, i, 0))],
        out_specs=pl.BlockSpec((1, 1, N), lambda c, i: (c, 0, 0)),
        out_shape=jax.ShapeDtypeStruct((2, 1, N), jnp.float32),
        compiler_params=pltpu.CompilerParams(
            dimension_semantics=("arbitrary", "arbitrary")),
    )(h)


def _xw_body(x_ref, w_ref, out_ref):
    out_ref[...] = jnp.dot(x_ref[...], w_ref[...],
                           preferred_element_type=jnp.float32)


def _xw(X, gcn_w):
    return pl.pallas_call(
        _xw_body,
        out_shape=jax.ShapeDtypeStruct((N, 128), jnp.float32),
    )(X, gcn_w)


def _final_body(h1_ref, xwTf_ref, xwTb_ref, csf_ref, csb_ref, gb_ref,
                w1t_ref, b1_ref, w2t_ref, b2_ref, out_ref):
    outs = []
    for c in range(2):
        csf = csf_ref[c:c + 1, :]
        dinv_f = jax.lax.rsqrt(1.0 + jnp.where(csf != 0, 1.0, 0.0))
        Yt = xwTf_ref[...] * dinv_f
        Zt = jnp.dot(Yt, h1_ref[c], preferred_element_type=jnp.float32)
        csb = csb_ref[c:c + 1, :]
        nz = jnp.where(csb != 0, 1.0, 0.0)
        dinv1 = jnp.where(csb != 0,
                          1.0 / jnp.where(csb != 0, csb, 1.0), 0.0)
        dinv_b = jax.lax.rsqrt(1.0 + nz)
        o = (Zt * (dinv_b * dinv1) + xwTb_ref[...] * (dinv_b * dinv_b)
             + gb_ref[...])
        outs.append(jnp.maximum(o, 0.0))
    xcat = jnp.concatenate(outs, axis=0)
    h = jnp.dot(w1t_ref[...], xcat, preferred_element_type=jnp.float32)
    h = jnp.maximum(h + b1_ref[...], 0.0)
    out_ref[...] = (jnp.dot(w2t_ref[...], h,
                            preferred_element_type=jnp.float32)
                    + b2_ref[...])


def _final(h1, xwT, cs1, gb, w1t, b1, w2t, b2):
    return pl.pallas_call(
        _final_body,
        grid=(N // BI,),
        in_specs=[
            pl.BlockSpec((2, N, BI), lambda i: (0, 0, i)),
            pl.BlockSpec((128, N), lambda i: (0, 0)),
            pl.BlockSpec((128, BI), lambda i: (0, i)),
            pl.BlockSpec((2, N), lambda i: (0, 0)),
            pl.BlockSpec((2, BI), lambda i: (0, i)),
            pl.BlockSpec((128, 1), lambda i: (0, 0)),
            pl.BlockSpec((128, 256), lambda i: (0, 0)),
            pl.BlockSpec((128, 1), lambda i: (0, 0)),
            pl.BlockSpec((128, 128), lambda i: (0, 0)),
            pl.BlockSpec((128, 1), lambda i: (0, 0)),
        ],
        out_specs=pl.BlockSpec((128, BI), lambda i: (0, i)),
        out_shape=jax.ShapeDtypeStruct((128, N), jnp.float32),
        compiler_params=pltpu.CompilerParams(
            dimension_semantics=("arbitrary",)),
    )(h1, xwT, xwT, cs1, cs1, gb, w1t, b1, w2t, b2)


def kernel(edge_indices, edge_values, X, conv_w1_0, conv_w2_0, conv_w1_1,
           gcn_w, gcn_b, lin1_w, lin1_b, lin2_w, lin2_b):
    F = jnp.concatenate([
        jax.nn.softmax(conv_w1_0, axis=1),
        jax.nn.softmax(conv_w2_0, axis=1),
        jax.nn.softmax(conv_w1_1, axis=1),
    ], axis=0)
    A = _build_dense(edge_indices, edge_values)
    P = _combos(F, A)
    H0 = _mm(P[0:2], P[2:4])
    cs0 = _colsum(H0)
    H1 = _mm(H0, P[4:6], cs=cs0)
    cs1 = _colsum(H1).reshape(2, N)
    xwT = _xw(X, gcn_w).T
    yT = _final(H1, xwT, cs1, gcn_b.reshape(128, 1),
                lin1_w.T, lin1_b.reshape(128, 1),
                lin2_w.T, lin2_b.reshape(128, 1))
    return yT.T


# speedup vs baseline: 1.2068x; 1.2068x over previous
import jax
import jax.numpy as jnp
from jax.experimental import pallas as pl
from jax.experimental.pallas import tpu as pltpu

N = 2048
BM = BN = 1024
BK = 512
KB = N // BK
BI = 512
BR = 128


def _build_dense(edge_indices, edge_values):
    def one(ei, ev):
        return jnp.zeros((N, N), dtype=ev.dtype).at[ei[0], ei[1]].add(ev)
    return jax.vmap(one)(edge_indices, edge_values)


def _combo_body(f_ref, a_ref, out_ref):
    for c in range(6):
        acc = f_ref[c, 0] * a_ref[0]
        for e in range(1, 5):
            acc = acc + f_ref[c, e] * a_ref[e]
        out_ref[c] = acc.astype(jnp.bfloat16)


def _combos(F, A):
    return pl.pallas_call(
        _combo_body,
        grid=(N // BR,),
        in_specs=[
            pl.BlockSpec(memory_space=pltpu.SMEM),
            pl.BlockSpec((5, BR, N), lambda i: (0, i, 0)),
        ],
        out_specs=pl.BlockSpec((6, BR, N), lambda i: (0, i, 0)),
        out_shape=jax.ShapeDtypeStruct((6, N, N), jnp.bfloat16),
        compiler_params=pltpu.CompilerParams(
            dimension_semantics=("arbitrary",)),
    )(F, A)


def _mm_plain_body(a_ref, b_ref, out_ref, acc_ref):
    i = pl.program_id(1)
    j = pl.program_id(2)
    k = pl.program_id(3)

    @pl.when(k == 0)
    def _():
        acc_ref[...] = jnp.zeros_like(acc_ref)

    acc_ref[...] += jnp.dot(a_ref[0], b_ref[0],
                            preferred_element_type=jnp.float32)

    @pl.when(k == KB - 1)
    def _():
        r = acc_ref[...]
        ir = jax.lax.broadcasted_iota(jnp.int32, (BM, BN), 0) + i * BM
        ic = jax.lax.broadcasted_iota(jnp.int32, (BM, BN), 1) + j * BN
        out_ref[0] = jnp.where(ir == ic, 0.0, r)


def _mm_scaled_body(cs_ref, a_ref, b_ref, out_ref, acc_ref):
    i = pl.program_id(1)
    j = pl.program_id(2)
    k = pl.program_id(3)

    @pl.when(k == 0)
    def _():
        acc_ref[...] = jnp.zeros_like(acc_ref)

    sc = cs_ref[0, 0]
    dinv = jnp.where(sc != 0, 1.0 / jnp.where(sc != 0, sc, 1.0), 0.0)
    acc_ref[...] += jnp.dot((a_ref[0] * dinv).astype(jnp.bfloat16),
                            b_ref[0], preferred_element_type=jnp.float32)

    @pl.when(k == KB - 1)
    def _():
        r = acc_ref[...]
        ir = jax.lax.broadcasted_iota(jnp.int32, (BM, BN), 0) + i * BM
        ic = jax.lax.broadcasted_iota(jnp.int32, (BM, BN), 1) + j * BN
        out_ref[0] = jnp.where(ir == ic, 0.0, r)


def _mm(a, b, cs=None):
    grid = (2, N // BM, N // BN, KB)
    specs = [
        pl.BlockSpec((1, BM, BK), lambda c, i, j, k: (c, i, k)),
        pl.BlockSpec((1, BK, BN), lambda c, i, j, k: (c, k, j)),
    ]
    args = [a, b]
    body = _mm_plain_body
    if cs is not None:
        specs.insert(0, pl.BlockSpec((1, 1, 1, BK),
                                     lambda c, i, j, k: (c, k, 0, 0)))
        args.insert(0, cs.reshape(2, KB, 1, BK))
        body = _mm_scaled_body
    return pl.pallas_call(
        body,
        grid=grid,
        in_specs=specs,
        out_specs=pl.BlockSpec((1, BM, BN), lambda c, i, j, k: (c, i, j)),
        out_shape=jax.ShapeDtypeStruct((2, N, N), jnp.float32),
        scratch_shapes=[pltpu.VMEM((BM, BN), jnp.float32)],
        compiler_params=pltpu.CompilerParams(
            dimension_semantics=("parallel", "parallel", "arbitrary",
                                 "arbitrary")),
    )(*args)


def _colsum_body(h_ref, out_ref):
    i = pl.program_id(1)

    @pl.when(i == 0)
    def _():
        out_ref[...] = jnp.zeros_like(out_ref)

    out_ref[0] += jnp.sum(h_ref[0], axis=0, keepdims=True)


def _colsum(h):
    blk = 256
    return pl.pallas_call(
        _colsum_body,
        grid=(2, N // blk),
        in_specs=[pl.BlockSpec((1, blk, N), lambda c, i: (c, i, 0))],
        out_specs=pl.BlockSpec((1, 1, N), lambda c, i: (c, 0, 0)),
        out_shape=jax.ShapeDtypeStruct((2, 1, N), jnp.float32),
        compiler_params=pltpu.CompilerParams(
            dimension_semantics=("arbitrary", "arbitrary")),
    )(h)


def _xw_body(x_ref, w_ref, out_ref):
    out_ref[...] = jnp.dot(x_ref[...], w_ref[...],
                           preferred_element_type=jnp.float32)


def _xw(X, gcn_w):
    return pl.pallas_call(
        _xw_body,
        out_shape=jax.ShapeDtypeStruct((N, 128), jnp.float32),
    )(X, gcn_w)


def _final_body(h1_ref, xwTf_ref, xwTb_ref, csf_ref, csb_ref, gb_ref,
                w1t_ref, b1_ref, w2t_ref, b2_ref, out_ref):
    outs = []
    for c in range(2):
        csf = csf_ref[c:c + 1, :]
        dinv_f = jax.lax.rsqrt(1.0 + jnp.where(csf != 0, 1.0, 0.0))
        Yt = xwTf_ref[...] * dinv_f
        Zt = jnp.dot(Yt, h1_ref[c], preferred_element_type=jnp.float32)
        csb = csb_ref[c:c + 1, :]
        nz = jnp.where(csb != 0, 1.0, 0.0)
        dinv1 = jnp.where(csb != 0,
                          1.0 / jnp.where(csb != 0, csb, 1.0), 0.0)
        dinv_b = jax.lax.rsqrt(1.0 + nz)
        o = (Zt * (dinv_b * dinv1) + xwTb_ref[...] * (dinv_b * dinv_b)
             + gb_ref[...])
        outs.append(jnp.maximum(o, 0.0))
    xcat = jnp.concatenate(outs, axis=0)
    h = jnp.dot(w1t_ref[...], xcat, preferred_element_type=jnp.float32)
    h = jnp.maximum(h + b1_ref[...], 0.0)
    out_ref[...] = (jnp.dot(w2t_ref[...], h,
                            preferred_element_type=jnp.float32)
                    + b2_ref[...])


def _final(h1, xwT, cs1, gb, w1t, b1, w2t, b2):
    return pl.pallas_call(
        _final_body,
        grid=(N // BI,),
        in_specs=[
            pl.BlockSpec((2, N, BI), lambda i: (0, 0, i)),
            pl.BlockSpec((128, N), lambda i: (0, 0)),
            pl.BlockSpec((128, BI), lambda i: (0, i)),
            pl.BlockSpec((2, N), lambda i: (0, 0)),
            pl.BlockSpec((2, BI), lambda i: (0, i)),
            pl.BlockSpec((128, 1), lambda i: (0, 0)),
            pl.BlockSpec((128, 256), lambda i: (0, 0)),
            pl.BlockSpec((128, 1), lambda i: (0, 0)),
            pl.BlockSpec((128, 128), lambda i: (0, 0)),
            pl.BlockSpec((128, 1), lambda i: (0, 0)),
        ],
        out_specs=pl.BlockSpec((128, BI), lambda i: (0, i)),
        out_shape=jax.ShapeDtypeStruct((128, N), jnp.float32),
        compiler_params=pltpu.CompilerParams(
            dimension_semantics=("arbitrary",)),
    )(h1, xwT, xwT, cs1, cs1, gb, w1t, b1, w2t, b2)


def kernel(edge_indices, edge_values, X, conv_w1_0, conv_w2_0, conv_w1_1,
           gcn_w, gcn_b, lin1_w, lin1_b, lin2_w, lin2_b):
    F = jnp.concatenate([
        jax.nn.softmax(conv_w1_0, axis=1),
        jax.nn.softmax(conv_w2_0, axis=1),
        jax.nn.softmax(conv_w1_1, axis=1),
    ], axis=0)
    A = _build_dense(edge_indices, edge_values)
    P = _combos(F, A)
    H0 = _mm(P[0:2], P[2:4])
    cs0 = _colsum(H0)
    H1 = _mm(H0, P[4:6], cs=cs0)
    cs1 = _colsum(H1).reshape(2, N)
    xwT = _xw(X, gcn_w).T
    yT = _final(H1, xwT, cs1, gcn_b.reshape(128, 1),
                lin1_w.T, lin1_b.reshape(128, 1),
                lin2_w.T, lin2_b.reshape(128, 1))
    return yT.T


# speedup vs baseline: 2.5756x; 2.1342x over previous
import functools

import jax
import jax.numpy as jnp
from jax import lax
from jax.experimental import pallas as pl
from jax.experimental.pallas import tpu as pltpu
from jax.experimental.pallas import tpu_sc as plsc

N = 2048
BM = BN = 1024
BK = 512
KB = N // BK
BI = 512
BR = 128


QW = 512
QWORDS = N * QW
SINK = QWORDS
_EDGES_PER_TILE = 65536 // 16


def _sc_body(ei_hbm, ev_hbm, out_hbm, rows_v, cols_v, vals_v, idx_v,
             zero_v, acc_sh):
    cid = lax.axis_index("c")
    sid = lax.axis_index("s")
    ept = _EDGES_PER_TILE

    def zinit(i, carry):
        zero_v[pl.ds(i * 16, 16)] = jnp.zeros((16,), jnp.float32)
        return carry
    lax.fori_loop(0, 1024, zinit, 0)

    lane8 = lax.iota(jnp.int32, 16) * 8

    for s in range(10):
        e = jnp.where(cid == 0, s // 4, (10 + s) // 4)
        q = jnp.where(cid == 0, s % 4, (10 + s) % 4)
        base = q * QW

        for z in range(4):
            pltpu.sync_copy(
                zero_v, acc_sh.at[pl.ds(sid * 65536 + z * 16384, 16384)])
        plsc.subcore_barrier()

        pltpu.sync_copy(ei_hbm.at[e, 0, pl.ds(sid * ept, ept)], rows_v)
        pltpu.sync_copy(ei_hbm.at[e, 1, pl.ds(sid * ept, ept)], cols_v)
        pltpu.sync_copy(ev_hbm.at[e, pl.ds(sid * ept, ept)], vals_v)

        def body(i, carry):
            r = rows_v[pl.ds(i * 16, 16)]
            c = cols_v[pl.ds(i * 16, 16)]
            m = (c >= base) & (c < base + QW)
            flat = r * QW + (c - base)
            idx_v[pl.ds(i * 16, 16)] = jnp.where(m, flat, SINK + lane8)
            return carry
        lax.fori_loop(0, ept // 16, body, 0)

        pltpu.sync_copy(vals_v, acc_sh.at[idx_v], add=True)
        plsc.subcore_barrier()

        pltpu.sync_copy(acc_sh.at[pl.ds(sid * 65536, 65536)],
                        out_hbm.at[e, q, sid])
        plsc.subcore_barrier()


def _build_dense(edge_indices, edge_values):
    mesh = plsc.VectorSubcoreMesh(core_axis_name="c", subcore_axis_name="s")
    k = functools.partial(
        pl.kernel,
        out_type=jax.ShapeDtypeStruct((5, 4, 16, 65536), jnp.float32),
        mesh=mesh,
        scratch_types=[
            pltpu.VMEM((_EDGES_PER_TILE,), jnp.int32),
            pltpu.VMEM((_EDGES_PER_TILE,), jnp.int32),
            pltpu.VMEM((_EDGES_PER_TILE,), jnp.float32),
            pltpu.VMEM((_EDGES_PER_TILE,), jnp.int32),
            pltpu.VMEM((16384,), jnp.float32),
            pltpu.VMEM_SHARED((QWORDS + 128,), jnp.float32),
        ],
    )(_sc_body)
    out = k(edge_indices, edge_values)
    return out.reshape(5, 4, N, QW)


def _combo_body(f_ref, a_ref, out_ref):
    for c in range(6):
        for q in range(4):
            acc = f_ref[c, 0] * a_ref[0, q]
            for e in range(1, 5):
                acc = acc + f_ref[c, e] * a_ref[e, q]
            out_ref[c, :, q * QW:(q + 1) * QW] = acc.astype(jnp.bfloat16)


def _combos(F, A):
    return pl.pallas_call(
        _combo_body,
        grid=(N // BR,),
        in_specs=[
            pl.BlockSpec(memory_space=pltpu.SMEM),
            pl.BlockSpec((5, 4, BR, QW), lambda i: (0, 0, i, 0)),
        ],
        out_specs=pl.BlockSpec((6, BR, N), lambda i: (0, i, 0)),
        out_shape=jax.ShapeDtypeStruct((6, N, N), jnp.bfloat16),
        compiler_params=pltpu.CompilerParams(
            dimension_semantics=("arbitrary",)),
    )(F, A)


def _mm_plain_body(a_ref, b_ref, out_ref, acc_ref):
    i = pl.program_id(1)
    j = pl.program_id(2)
    k = pl.program_id(3)

    @pl.when(k == 0)
    def _():
        acc_ref[...] = jnp.zeros_like(acc_ref)

    acc_ref[...] += jnp.dot(a_ref[0], b_ref[0],
                            preferred_element_type=jnp.float32)

    @pl.when(k == KB - 1)
    def _():
        r = acc_ref[...]
        ir = jax.lax.broadcasted_iota(jnp.int32, (BM, BN), 0) + i * BM
        ic = jax.lax.broadcasted_iota(jnp.int32, (BM, BN), 1) + j * BN
        out_ref[0] = jnp.where(ir == ic, 0.0, r)


def _mm_scaled_body(cs_ref, a_ref, b_ref, out_ref, acc_ref):
    i = pl.program_id(1)
    j = pl.program_id(2)
    k = pl.program_id(3)

    @pl.when(k == 0)
    def _():
        acc_ref[...] = jnp.zeros_like(acc_ref)

    sc = cs_ref[0, 0]
    dinv = jnp.where(sc != 0, 1.0 / jnp.where(sc != 0, sc, 1.0), 0.0)
    acc_ref[...] += jnp.dot((a_ref[0] * dinv).astype(jnp.bfloat16),
                            b_ref[0], preferred_element_type=jnp.float32)

    @pl.when(k == KB - 1)
    def _():
        r = acc_ref[...]
        ir = jax.lax.broadcasted_iota(jnp.int32, (BM, BN), 0) + i * BM
        ic = jax.lax.broadcasted_iota(jnp.int32, (BM, BN), 1) + j * BN
        out_ref[0] = jnp.where(ir == ic, 0.0, r)


def _mm(a, b, cs=None):
    grid = (2, N // BM, N // BN, KB)
    specs = [
        pl.BlockSpec((1, BM, BK), lambda c, i, j, k: (c, i, k)),
        pl.BlockSpec((1, BK, BN), lambda c, i, j, k: (c, k, j)),
    ]
    args = [a, b]
    body = _mm_plain_body
    if cs is not None:
        specs.insert(0, pl.BlockSpec((1, 1, 1, BK),
                                     lambda c, i, j, k: (c, k, 0, 0)))
        args.insert(0, cs.reshape(2, KB, 1, BK))
        body = _mm_scaled_body
    return pl.pallas_call(
        body,
        grid=grid,
        in_specs=specs,
        out_specs=pl.BlockSpec((1, BM, BN), lambda c, i, j, k: (c, i, j)),
        out_shape=jax.ShapeDtypeStruct((2, N, N), jnp.float32),
        scratch_shapes=[pltpu.VMEM((BM, BN), jnp.float32)],
        compiler_params=pltpu.CompilerParams(
            dimension_semantics=("parallel", "parallel", "arbitrary",
                                 "arbitrary")),
    )(*args)


def _colsum_body(h_ref, out_ref):
    i = pl.program_id(1)

    @pl.when(i == 0)
    def _():
        out_ref[...] = jnp.zeros_like(out_ref)

    out_ref[0] += jnp.sum(h_ref[0], axis=0, keepdims=True)


def _colsum(h):
    blk = 256
    return pl.pallas_call(
        _colsum_body,
        grid=(2, N // blk),
        in_specs=[pl.BlockSpec((1, blk, N), lambda c, i: (c, i, 0))],
        out_specs=pl.BlockSpec((1, 1, N), lambda c, i: (c, 0, 0)),
        out_shape=jax.ShapeDtypeStruct((2, 1, N), jnp.float32),
        compiler_params=pltpu.CompilerParams(
            dimension_semantics=("arbitrary", "arbitrary")),
    )(h)


def _xw_body(x_ref, w_ref, out_ref):
    out_ref[...] = jnp.dot(x_ref[...], w_ref[...],
                           preferred_element_type=jnp.float32)


def _xw(X, gcn_w):
    return pl.pallas_call(
        _xw_body,
        out_shape=jax.ShapeDtypeStruct((N, 128), jnp.float32),
    )(X, gcn_w)


def _final_body(h1_ref, xwTf_ref, xwTb_ref, csf_ref, csb_ref, gb_ref,
                w1t_ref, b1_ref, w2t_ref, b2_ref, out_ref):
    outs = []
    for c in range(2):
        csf = csf_ref[c:c + 1, :]
        dinv_f = jax.lax.rsqrt(1.0 + jnp.where(csf != 0, 1.0, 0.0))
        Yt = xwTf_ref[...] * dinv_f
        Zt = jnp.dot(Yt, h1_ref[c], preferred_element_type=jnp.float32)
        csb = csb_ref[c:c + 1, :]
        nz = jnp.where(csb != 0, 1.0, 0.0)
        dinv1 = jnp.where(csb != 0,
                          1.0 / jnp.where(csb != 0, csb, 1.0), 0.0)
        dinv_b = jax.lax.rsqrt(1.0 + nz)
        o = (Zt * (dinv_b * dinv1) + xwTb_ref[...] * (dinv_b * dinv_b)
             + gb_ref[...])
        outs.append(jnp.maximum(o, 0.0))
    xcat = jnp.concatenate(outs, axis=0)
    h = jnp.dot(w1t_ref[...], xcat, preferred_element_type=jnp.float32)
    h = jnp.maximum(h + b1_ref[...], 0.0)
    out_ref[...] = (jnp.dot(w2t_ref[...], h,
                            preferred_element_type=jnp.float32)
                    + b2_ref[...])


def _final(h1, xwT, cs1, gb, w1t, b1, w2t, b2):
    return pl.pallas_call(
        _final_body,
        grid=(N // BI,),
        in_specs=[
            pl.BlockSpec((2, N, BI), lambda i: (0, 0, i)),
            pl.BlockSpec((128, N), lambda i: (0, 0)),
            pl.BlockSpec((128, BI), lambda i: (0, i)),
            pl.BlockSpec((2, N), lambda i: (0, 0)),
            pl.BlockSpec((2, BI), lambda i: (0, i)),
            pl.BlockSpec((128, 1), lambda i: (0, 0)),
            pl.BlockSpec((128, 256), lambda i: (0, 0)),
            pl.BlockSpec((128, 1), lambda i: (0, 0)),
            pl.BlockSpec((128, 128), lambda i: (0, 0)),
            pl.BlockSpec((128, 1), lambda i: (0, 0)),
        ],
        out_specs=pl.BlockSpec((128, BI), lambda i: (0, i)),
        out_shape=jax.ShapeDtypeStruct((128, N), jnp.float32),
        compiler_params=pltpu.CompilerParams(
            dimension_semantics=("arbitrary",)),
    )(h1, xwT, xwT, cs1, cs1, gb, w1t, b1, w2t, b2)


def kernel(edge_indices, edge_values, X, conv_w1_0, conv_w2_0, conv_w1_1,
           gcn_w, gcn_b, lin1_w, lin1_b, lin2_w, lin2_b):
    F = jnp.concatenate([
        jax.nn.softmax(conv_w1_0, axis=1),
        jax.nn.softmax(conv_w2_0, axis=1),
        jax.nn.softmax(conv_w1_1, axis=1),
    ], axis=0)
    A = _build_dense(edge_indices, edge_values)
    P = _combos(F, A)
    H0 = _mm(P[0:2], P[2:4])
    cs0 = _colsum(H0)
    H1 = _mm(H0, P[4:6], cs=cs0)
    cs1 = _colsum(H1).reshape(2, N)
    xwT = _xw(X, gcn_w).T
    yT = _final(H1, xwT, cs1, gcn_b.reshape(128, 1),
                lin1_w.T, lin1_b.reshape(128, 1),
                lin2_w.T, lin2_b.reshape(128, 1))
    return yT.T


# speedup vs baseline: 2.7997x; 1.0870x over previous
import functools

import jax
import jax.numpy as jnp
from jax import lax
from jax.experimental import pallas as pl
from jax.experimental.pallas import tpu as pltpu
from jax.experimental.pallas import tpu_sc as plsc

N = 2048
BM = BN = 1024
BK = 512
KB = N // BK
BI = 512
BR = 128


QW = 512
QWORDS = N * QW
SINK = QWORDS
_EDGES_PER_TILE = 65536 // 16


def _sc_body(ei_hbm, ev_hbm, out_hbm, rows_v, cols_v, vals_v, idx_v,
             zero_v, acc_sh):
    cid = lax.axis_index("c")
    sid = lax.axis_index("s")
    ept = _EDGES_PER_TILE

    def zinit(i, carry):
        zero_v[pl.ds(i * 16, 16)] = jnp.zeros((16,), jnp.float32)
        return carry
    lax.fori_loop(0, 1024, zinit, 0)

    lane8 = lax.iota(jnp.int32, 16) * 8

    for s in range(10):
        e = jnp.where(cid == 0, s // 4, (10 + s) // 4)
        q = jnp.where(cid == 0, s % 4, (10 + s) % 4)
        base = q * QW

        for z in range(4):
            pltpu.sync_copy(
                zero_v, acc_sh.at[pl.ds(sid * 65536 + z * 16384, 16384)])
        plsc.subcore_barrier()

        pltpu.sync_copy(ei_hbm.at[e, 0, pl.ds(sid * ept, ept)], rows_v)
        pltpu.sync_copy(ei_hbm.at[e, 1, pl.ds(sid * ept, ept)], cols_v)
        pltpu.sync_copy(ev_hbm.at[e, pl.ds(sid * ept, ept)], vals_v)

        def body(i, carry):
            r = rows_v[pl.ds(i * 16, 16)]
            c = cols_v[pl.ds(i * 16, 16)]
            m = (c >= base) & (c < base + QW)
            flat = r * QW + (c - base)
            idx_v[pl.ds(i * 16, 16)] = jnp.where(m, flat, SINK + lane8)
            return carry
        lax.fori_loop(0, ept // 16, body, 0)

        pltpu.sync_copy(vals_v, acc_sh.at[idx_v], add=True)
        plsc.subcore_barrier()

        pltpu.sync_copy(acc_sh.at[pl.ds(sid * 65536, 65536)],
                        out_hbm.at[e, q, sid])
        plsc.subcore_barrier()


def _build_dense(edge_indices, edge_values):
    mesh = plsc.VectorSubcoreMesh(core_axis_name="c", subcore_axis_name="s")
    k = functools.partial(
        pl.kernel,
        out_type=jax.ShapeDtypeStruct((5, 4, 16, 65536), jnp.float32),
        mesh=mesh,
        scratch_types=[
            pltpu.VMEM((_EDGES_PER_TILE,), jnp.int32),
            pltpu.VMEM((_EDGES_PER_TILE,), jnp.int32),
            pltpu.VMEM((_EDGES_PER_TILE,), jnp.float32),
            pltpu.VMEM((_EDGES_PER_TILE,), jnp.int32),
            pltpu.VMEM((16384,), jnp.float32),
            pltpu.VMEM_SHARED((QWORDS + 128,), jnp.float32),
        ],
    )(_sc_body)
    out = k(edge_indices, edge_values)
    return out.reshape(5, 4, N, QW)


def _combo_body(f_ref, a_ref, out_ref):
    for c in range(6):
        for q in range(4):
            acc = f_ref[c, 0] * a_ref[0, q]
            for e in range(1, 5):
                acc = acc + f_ref[c, e] * a_ref[e, q]
            out_ref[c, :, q * QW:(q + 1) * QW] = acc.astype(jnp.bfloat16)


def _combos(F, A):
    return pl.pallas_call(
        _combo_body,
        grid=(N // BR,),
        in_specs=[
            pl.BlockSpec(memory_space=pltpu.SMEM),
            pl.BlockSpec((5, 4, BR, QW), lambda i: (0, 0, i, 0)),
        ],
        out_specs=pl.BlockSpec((6, BR, N), lambda i: (0, i, 0)),
        out_shape=jax.ShapeDtypeStruct((6, N, N), jnp.bfloat16),
        compiler_params=pltpu.CompilerParams(
            dimension_semantics=("arbitrary",)),
    )(F, A)


def _mm_plain_body(a_ref, b_ref, out_ref, cs_ref, acc_ref):
    i = pl.program_id(2)
    j = pl.program_id(1)
    k = pl.program_id(3)

    @pl.when(k == 0)
    def _():
        acc_ref[...] = jnp.zeros_like(acc_ref)

    acc_ref[...] += jnp.dot(a_ref[0], b_ref[0],
                            preferred_element_type=jnp.float32)

    @pl.when(k == KB - 1)
    def _():
        r = acc_ref[...]
        ir = jax.lax.broadcasted_iota(jnp.int32, (BM, BN), 0) + i * BM
        ic = jax.lax.broadcasted_iota(jnp.int32, (BM, BN), 1) + j * BN
        r = jnp.where(ir == ic, 0.0, r)
        out_ref[0] = r.astype(jnp.bfloat16)
        part = jnp.sum(r, axis=0, keepdims=True)

        @pl.when(i == 0)
        def _():
            cs_ref[0, 0] = part

        @pl.when(i != 0)
        def _():
            cs_ref[0, 0] += part


def _mm_scaled_body(csin_ref, a_ref, b_ref, out_ref, cs_ref, acc_ref):
    i = pl.program_id(2)
    j = pl.program_id(1)
    k = pl.program_id(3)

    @pl.when(k == 0)
    def _():
        acc_ref[...] = jnp.zeros_like(acc_ref)

    sc = csin_ref[0, 0]
    dinv = jnp.where(sc != 0, 1.0 / jnp.where(sc != 0, sc, 1.0), 0.0)
    acc_ref[...] += jnp.dot((a_ref[0] * dinv).astype(jnp.bfloat16),
                            b_ref[0], preferred_element_type=jnp.float32)

    @pl.when(k == KB - 1)
    def _():
        r = acc_ref[...]
        ir = jax.lax.broadcasted_iota(jnp.int32, (BM, BN), 0) + i * BM
        ic = jax.lax.broadcasted_iota(jnp.int32, (BM, BN), 1) + j * BN
        r = jnp.where(ir == ic, 0.0, r)
        out_ref[0] = r.astype(jnp.bfloat16)
        part = jnp.sum(r, axis=0, keepdims=True)

        @pl.when(i == 0)
        def _():
            cs_ref[0, 0] = part

        @pl.when(i != 0)
        def _():
            cs_ref[0, 0] += part


def _mm(a, b, cs=None):
    grid = (2, N // BN, N // BM, KB)
    specs = [
        pl.BlockSpec((1, BM, BK), lambda c, j, i, k: (c, i, k)),
        pl.BlockSpec((1, BK, BN), lambda c, j, i, k: (c, k, j)),
    ]
    args = [a, b]
    body = _mm_plain_body
    if cs is not None:
        specs.insert(0, pl.BlockSpec((1, 1, 1, BK),
                                     lambda c, j, i, k: (c, k, 0, 0)))
        args.insert(0, cs.reshape(2, KB, 1, BK))
        body = _mm_scaled_body
    return pl.pallas_call(
        body,
        grid=grid,
        in_specs=specs,
        out_specs=[
            pl.BlockSpec((1, BM, BN), lambda c, j, i, k: (c, i, j)),
            pl.BlockSpec((1, 1, 1, BN), lambda c, j, i, k: (c, j, 0, 0)),
        ],
        out_shape=[
            jax.ShapeDtypeStruct((2, N, N), jnp.bfloat16),
            jax.ShapeDtypeStruct((2, N // BN, 1, BN), jnp.float32),
        ],
        scratch_shapes=[pltpu.VMEM((BM, BN), jnp.float32)],
        compiler_params=pltpu.CompilerParams(
            dimension_semantics=("parallel", "parallel", "arbitrary",
                                 "arbitrary")),
    )(*args)


def _xw_body(x_ref, w_ref, out_ref):
    out_ref[...] = jnp.dot(x_ref[...], w_ref[...],
                           preferred_element_type=jnp.float32)


def _xw(X, gcn_w):
    return pl.pallas_call(
        _xw_body,
        out_shape=jax.ShapeDtypeStruct((N, 128), jnp.float32),
    )(X, gcn_w)


def _final_body(h1_ref, xwTf_ref, xwTb_ref, csf_ref, csb_ref, gb_ref,
                w1t_ref, b1_ref, w2t_ref, b2_ref, out_ref):
    outs = []
    for c in range(2):
        csf = csf_ref[c:c + 1, :]
        dinv_f = jax.lax.rsqrt(1.0 + jnp.where(csf != 0, 1.0, 0.0))
        Yt = (xwTf_ref[...] * dinv_f).astype(jnp.bfloat16)
        Zt = jnp.dot(Yt, h1_ref[c], preferred_element_type=jnp.float32)
        csb = csb_ref[c:c + 1, :]
        nz = jnp.where(csb != 0, 1.0, 0.0)
        dinv1 = jnp.where(csb != 0,
                          1.0 / jnp.where(csb != 0, csb, 1.0), 0.0)
        dinv_b = jax.lax.rsqrt(1.0 + nz)
        o = (Zt * (dinv_b * dinv1) + xwTb_ref[...] * (dinv_b * dinv_b)
             + gb_ref[...])
        outs.append(jnp.maximum(o, 0.0))
    xcat = jnp.concatenate(outs, axis=0)
    h = jnp.dot(w1t_ref[...], xcat, preferred_element_type=jnp.float32)
    h = jnp.maximum(h + b1_ref[...], 0.0)
    out_ref[...] = (jnp.dot(w2t_ref[...], h,
                            preferred_element_type=jnp.float32)
                    + b2_ref[...])


def _final(h1, xwT, cs1, gb, w1t, b1, w2t, b2):
    return pl.pallas_call(
        _final_body,
        grid=(N // BI,),
        in_specs=[
            pl.BlockSpec((2, N, BI), lambda i: (0, 0, i)),
            pl.BlockSpec((128, N), lambda i: (0, 0)),
            pl.BlockSpec((128, BI), lambda i: (0, i)),
            pl.BlockSpec((2, N), lambda i: (0, 0)),
            pl.BlockSpec((2, BI), lambda i: (0, i)),
            pl.BlockSpec((128, 1), lambda i: (0, 0)),
            pl.BlockSpec((128, 256), lambda i: (0, 0)),
            pl.BlockSpec((128, 1), lambda i: (0, 0)),
            pl.BlockSpec((128, 128), lambda i: (0, 0)),
            pl.BlockSpec((128, 1), lambda i: (0, 0)),
        ],
        out_specs=pl.BlockSpec((128, BI), lambda i: (0, i)),
        out_shape=jax.ShapeDtypeStruct((128, N), jnp.float32),
        compiler_params=pltpu.CompilerParams(
            dimension_semantics=("arbitrary",)),
    )(h1, xwT, xwT, cs1, cs1, gb, w1t, b1, w2t, b2)


def kernel(edge_indices, edge_values, X, conv_w1_0, conv_w2_0, conv_w1_1,
           gcn_w, gcn_b, lin1_w, lin1_b, lin2_w, lin2_b):
    F = jnp.concatenate([
        jax.nn.softmax(conv_w1_0, axis=1),
        jax.nn.softmax(conv_w2_0, axis=1),
        jax.nn.softmax(conv_w1_1, axis=1),
    ], axis=0)
    A = _build_dense(edge_indices, edge_values)
    P = _combos(F, A)
    H0, cs0 = _mm(P[0:2], P[2:4])
    H1, cs1 = _mm(H0, P[4:6], cs=cs0)
    cs1 = cs1.reshape(2, N)
    xwT = _xw(X, gcn_w).T
    yT = _final(H1, xwT, cs1, gcn_b.reshape(128, 1),
                lin1_w.T, lin1_b.reshape(128, 1),
                lin2_w.T, lin2_b.reshape(128, 1))
    return yT.T


# speedup vs baseline: 2.8092x; 1.0034x over previous
import functools

import jax
import jax.numpy as jnp
from jax import lax
from jax.experimental import pallas as pl
from jax.experimental.pallas import tpu as pltpu
from jax.experimental.pallas import tpu_sc as plsc

N = 2048
BM = BN = 1024
BK = 512
KB = N // BK
BI = 512
BR = 128


QW = 512
QWORDS = N * QW
SINK = QWORDS
_EDGES_PER_TILE = 65536 // 16


def _sc_body(ei_hbm, ev_hbm, out_hbm, rows_v, cols_v, vals_v, idx_v,
             zero_v, acc_sh):
    cid = lax.axis_index("c")
    sid = lax.axis_index("s")
    ept = _EDGES_PER_TILE

    def zinit(i, carry):
        zero_v[pl.ds(i * 16, 16)] = jnp.zeros((16,), jnp.float32)
        return carry
    lax.fori_loop(0, 1024, zinit, 0)

    lane8 = lax.iota(jnp.int32, 16) * 8

    def slice_body(s, carry):
        slice_id = cid * 10 + s
        e = slice_id // 4
        q = slice_id % 4
        base = q * QW

        pltpu.sync_copy(ei_hbm.at[e, 0, pl.ds(sid * ept, ept)], rows_v)
        pltpu.sync_copy(ei_hbm.at[e, 1, pl.ds(sid * ept, ept)], cols_v)
        pltpu.sync_copy(ev_hbm.at[e, pl.ds(sid * ept, ept)], vals_v)

        def body(i, c2):
            r = rows_v[pl.ds(i * 16, 16)]
            c = cols_v[pl.ds(i * 16, 16)]
            m = (c >= base) & (c < base + QW)
            flat = r * QW + (c - base)
            idx_v[pl.ds(i * 16, 16)] = jnp.where(m, flat, SINK + lane8)
            return c2
        lax.fori_loop(0, ept // 16, body, 0)

        for z in range(4):
            pltpu.sync_copy(
                zero_v, acc_sh.at[pl.ds(sid * 65536 + z * 16384, 16384)])
        plsc.subcore_barrier()

        pltpu.sync_copy(vals_v, acc_sh.at[idx_v], add=True)
        plsc.subcore_barrier()

        pltpu.sync_copy(acc_sh.at[pl.ds(sid * 65536, 65536)],
                        out_hbm.at[e, q, sid])
        return carry

    lax.fori_loop(0, 10, slice_body, 0)


def _build_dense(edge_indices, edge_values):
    mesh = plsc.VectorSubcoreMesh(core_axis_name="c", subcore_axis_name="s")
    k = functools.partial(
        pl.kernel,
        out_type=jax.ShapeDtypeStruct((5, 4, 16, 65536), jnp.float32),
        mesh=mesh,
        scratch_types=[
            pltpu.VMEM((_EDGES_PER_TILE,), jnp.int32),
            pltpu.VMEM((_EDGES_PER_TILE,), jnp.int32),
            pltpu.VMEM((_EDGES_PER_TILE,), jnp.float32),
            pltpu.VMEM((_EDGES_PER_TILE,), jnp.int32),
            pltpu.VMEM((16384,), jnp.float32),
            pltpu.VMEM_SHARED((QWORDS + 128,), jnp.float32),
        ],
    )(_sc_body)
    out = k(edge_indices, edge_values)
    return out.reshape(5, 4, N, QW)


def _combo_body(f_ref, a_ref, out_ref):
    for c in range(6):
        for q in range(4):
            acc = f_ref[c, 0] * a_ref[0, q]
            for e in range(1, 5):
                acc = acc + f_ref[c, e] * a_ref[e, q]
            out_ref[c, :, q * QW:(q + 1) * QW] = acc.astype(jnp.bfloat16)


def _combos(F, A):
    return pl.pallas_call(
        _combo_body,
        grid=(N // BR,),
        in_specs=[
            pl.BlockSpec(memory_space=pltpu.SMEM),
            pl.BlockSpec((5, 4, BR, QW), lambda i: (0, 0, i, 0)),
        ],
        out_specs=pl.BlockSpec((6, BR, N), lambda i: (0, i, 0)),
        out_shape=jax.ShapeDtypeStruct((6, N, N), jnp.bfloat16),
        compiler_params=pltpu.CompilerParams(
            dimension_semantics=("arbitrary",)),
    )(F, A)


def _mm_plain_body(a_ref, b_ref, out_ref, cs_ref, acc_ref):
    i = pl.program_id(2)
    j = pl.program_id(1)
    k = pl.program_id(3)

    @pl.when(k == 0)
    def _():
        acc_ref[...] = jnp.zeros_like(acc_ref)

    acc_ref[...] += jnp.dot(a_ref[0], b_ref[0],
                            preferred_element_type=jnp.float32)

    @pl.when(k == KB - 1)
    def _():
        r = acc_ref[...]
        ir = jax.lax.broadcasted_iota(jnp.int32, (BM, BN), 0) + i * BM
        ic = jax.lax.broadcasted_iota(jnp.int32, (BM, BN), 1) + j * BN
        r = jnp.where(ir == ic, 0.0, r)
        out_ref[0] = r.astype(jnp.bfloat16)
        part = jnp.sum(r, axis=0, keepdims=True)

        @pl.when(i == 0)
        def _():
            cs_ref[0, 0] = part

        @pl.when(i != 0)
        def _():
            cs_ref[0, 0] += part


def _mm_scaled_body(csin_ref, a_ref, b_ref, out_ref, cs_ref, acc_ref):
    i = pl.program_id(2)
    j = pl.program_id(1)
    k = pl.program_id(3)

    @pl.when(k == 0)
    def _():
        acc_ref[...] = jnp.zeros_like(acc_ref)

    sc = csin_ref[0, 0]
    dinv = jnp.where(sc != 0, 1.0 / jnp.where(sc != 0, sc, 1.0), 0.0)
    acc_ref[...] += jnp.dot((a_ref[0] * dinv).astype(jnp.bfloat16),
                            b_ref[0], preferred_element_type=jnp.float32)

    @pl.when(k == KB - 1)
    def _():
        r = acc_ref[...]
        ir = jax.lax.broadcasted_iota(jnp.int32, (BM, BN), 0) + i * BM
        ic = jax.lax.broadcasted_iota(jnp.int32, (BM, BN), 1) + j * BN
        r = jnp.where(ir == ic, 0.0, r)
        out_ref[0] = r.astype(jnp.bfloat16)
        part = jnp.sum(r, axis=0, keepdims=True)

        @pl.when(i == 0)
        def _():
            cs_ref[0, 0] = part

        @pl.when(i != 0)
        def _():
            cs_ref[0, 0] += part


def _mm(a, b, cs=None):
    grid = (2, N // BN, N // BM, KB)
    specs = [
        pl.BlockSpec((1, BM, BK), lambda c, j, i, k: (c, i, k)),
        pl.BlockSpec((1, BK, BN), lambda c, j, i, k: (c, k, j)),
    ]
    args = [a, b]
    body = _mm_plain_body
    if cs is not None:
        specs.insert(0, pl.BlockSpec((1, 1, 1, BK),
                                     lambda c, j, i, k: (c, k, 0, 0)))
        args.insert(0, cs.reshape(2, KB, 1, BK))
        body = _mm_scaled_body
    return pl.pallas_call(
        body,
        grid=grid,
        in_specs=specs,
        out_specs=[
            pl.BlockSpec((1, BM, BN), lambda c, j, i, k: (c, i, j)),
            pl.BlockSpec((1, 1, 1, BN), lambda c, j, i, k: (c, j, 0, 0)),
        ],
        out_shape=[
            jax.ShapeDtypeStruct((2, N, N), jnp.bfloat16),
            jax.ShapeDtypeStruct((2, N // BN, 1, BN), jnp.float32),
        ],
        scratch_shapes=[pltpu.VMEM((BM, BN), jnp.float32)],
        compiler_params=pltpu.CompilerParams(
            dimension_semantics=("parallel", "parallel", "arbitrary",
                                 "arbitrary")),
    )(*args)


def _xw_body(x_ref, w_ref, out_ref):
    out_ref[...] = jnp.dot(x_ref[...], w_ref[...],
                           preferred_element_type=jnp.float32)


def _xw(X, gcn_w):
    return pl.pallas_call(
        _xw_body,
        out_shape=jax.ShapeDtypeStruct((N, 128), jnp.float32),
    )(X, gcn_w)


def _final_body(h1_ref, xwTf_ref, xwTb_ref, csf_ref, csb_ref, gb_ref,
                w1t_ref, b1_ref, w2t_ref, b2_ref, out_ref):
    outs = []
    for c in range(2):
        csf = csf_ref[c:c + 1, :]
        dinv_f = jax.lax.rsqrt(1.0 + jnp.where(csf != 0, 1.0, 0.0))
        Yt = (xwTf_ref[...] * dinv_f).astype(jnp.bfloat16)
        Zt = jnp.dot(Yt, h1_ref[c], preferred_element_type=jnp.float32)
        csb = csb_ref[c:c + 1, :]
        nz = jnp.where(csb != 0, 1.0, 0.0)
        dinv1 = jnp.where(csb != 0,
                          1.0 / jnp.where(csb != 0, csb, 1.0), 0.0)
        dinv_b = jax.lax.rsqrt(1.0 + nz)
        o = (Zt * (dinv_b * dinv1) + xwTb_ref[...] * (dinv_b * dinv_b)
             + gb_ref[...])
        outs.append(jnp.maximum(o, 0.0))
    xcat = jnp.concatenate(outs, axis=0)
    h = jnp.dot(w1t_ref[...], xcat, preferred_element_type=jnp.float32)
    h = jnp.maximum(h + b1_ref[...], 0.0)
    out_ref[...] = (jnp.dot(w2t_ref[...], h,
                            preferred_element_type=jnp.float32)
                    + b2_ref[...])


def _final(h1, xwT, cs1, gb, w1t, b1, w2t, b2):
    return pl.pallas_call(
        _final_body,
        grid=(N // BI,),
        in_specs=[
            pl.BlockSpec((2, N, BI), lambda i: (0, 0, i)),
            pl.BlockSpec((128, N), lambda i: (0, 0)),
            pl.BlockSpec((128, BI), lambda i: (0, i)),
            pl.BlockSpec((2, N), lambda i: (0, 0)),
            pl.BlockSpec((2, BI), lambda i: (0, i)),
            pl.BlockSpec((128, 1), lambda i: (0, 0)),
            pl.BlockSpec((128, 256), lambda i: (0, 0)),
            pl.BlockSpec((128, 1), lambda i: (0, 0)),
            pl.BlockSpec((128, 128), lambda i: (0, 0)),
            pl.BlockSpec((128, 1), lambda i: (0, 0)),
        ],
        out_specs=pl.BlockSpec((128, BI), lambda i: (0, i)),
        out_shape=jax.ShapeDtypeStruct((128, N), jnp.float32),
        compiler_params=pltpu.CompilerParams(
            dimension_semantics=("arbitrary",)),
    )(h1, xwT, xwT, cs1, cs1, gb, w1t, b1, w2t, b2)


def kernel(edge_indices, edge_values, X, conv_w1_0, conv_w2_0, conv_w1_1,
           gcn_w, gcn_b, lin1_w, lin1_b, lin2_w, lin2_b):
    F = jnp.concatenate([
        jax.nn.softmax(conv_w1_0, axis=1),
        jax.nn.softmax(conv_w2_0, axis=1),
        jax.nn.softmax(conv_w1_1, axis=1),
    ], axis=0)
    A = _build_dense(edge_indices, edge_values)
    P = _combos(F, A)
    H0, cs0 = _mm(P[0:2], P[2:4])
    H1, cs1 = _mm(H0, P[4:6], cs=cs0)
    cs1 = cs1.reshape(2, N)
    xwT = _xw(X, gcn_w).T
    yT = _final(H1, xwT, cs1, gcn_b.reshape(128, 1),
                lin1_w.T, lin1_b.reshape(128, 1),
                lin2_w.T, lin2_b.reshape(128, 1))
    return yT.T


# speedup vs baseline: 2.8767x; 1.0240x over previous
import functools

import jax
import jax.numpy as jnp
from jax import lax
from jax.experimental import pallas as pl
from jax.experimental.pallas import tpu as pltpu
from jax.experimental.pallas import tpu_sc as plsc

N = 2048
BM = BN = 1024
BK = 1024
KB = N // BK
BI = 512
BR = 64


QW = 512
QWORDS = N * QW
SINK = QWORDS
_EDGES_PER_TILE = 65536 // 16


def _sc_body(ei_hbm, ev_hbm, out_hbm, rows_v, cols_v, vals_v, idx_v,
             zero_v, acc_sh):
    cid = lax.axis_index("c")
    sid = lax.axis_index("s")
    ept = _EDGES_PER_TILE

    def zinit(i, carry):
        zero_v[pl.ds(i * 16, 16)] = jnp.zeros((16,), jnp.float32)
        return carry
    lax.fori_loop(0, 1024, zinit, 0)

    lane8 = lax.iota(jnp.int32, 16) * 8

    def slice_body(s, carry):
        slice_id = cid * 10 + s
        e = slice_id // 4
        q = slice_id % 4
        base = q * QW

        pltpu.sync_copy(ei_hbm.at[e, 0, pl.ds(sid * ept, ept)], rows_v)
        pltpu.sync_copy(ei_hbm.at[e, 1, pl.ds(sid * ept, ept)], cols_v)
        pltpu.sync_copy(ev_hbm.at[e, pl.ds(sid * ept, ept)], vals_v)

        def body(i, c2):
            r = rows_v[pl.ds(i * 16, 16)]
            c = cols_v[pl.ds(i * 16, 16)]
            m = (c >= base) & (c < base + QW)
            flat = r * QW + (c - base)
            idx_v[pl.ds(i * 16, 16)] = jnp.where(m, flat, SINK + lane8)
            return c2
        lax.fori_loop(0, ept // 16, body, 0)

        for z in range(4):
            pltpu.sync_copy(
                zero_v, acc_sh.at[pl.ds(sid * 65536 + z * 16384, 16384)])
        plsc.subcore_barrier()

        pltpu.sync_copy(vals_v, acc_sh.at[idx_v], add=True)
        plsc.subcore_barrier()

        pltpu.sync_copy(acc_sh.at[pl.ds(sid * 65536, 65536)],
                        out_hbm.at[e, q, sid])
        return carry

    lax.fori_loop(0, 10, slice_body, 0)


def _build_dense(edge_indices, edge_values):
    mesh = plsc.VectorSubcoreMesh(core_axis_name="c", subcore_axis_name="s")
    k = functools.partial(
        pl.kernel,
        out_type=jax.ShapeDtypeStruct((5, 4, 16, 65536), jnp.float32),
        mesh=mesh,
        scratch_types=[
            pltpu.VMEM((_EDGES_PER_TILE,), jnp.int32),
            pltpu.VMEM((_EDGES_PER_TILE,), jnp.int32),
            pltpu.VMEM((_EDGES_PER_TILE,), jnp.float32),
            pltpu.VMEM((_EDGES_PER_TILE,), jnp.int32),
            pltpu.VMEM((16384,), jnp.float32),
            pltpu.VMEM_SHARED((QWORDS + 128,), jnp.float32),
        ],
    )(_sc_body)
    out = k(edge_indices, edge_values)
    return out.reshape(5, 4, N, QW)


def _combo_body(f_ref, a_ref, out_ref):
    for c in range(6):
        for q in range(4):
            acc = f_ref[c, 0] * a_ref[0, q]
            for e in range(1, 5):
                acc = acc + f_ref[c, e] * a_ref[e, q]
            out_ref[c, :, q * QW:(q + 1) * QW] = acc.astype(jnp.bfloat16)


def _combos(F, A):
    return pl.pallas_call(
        _combo_body,
        grid=(N // BR,),
        in_specs=[
            pl.BlockSpec(memory_space=pltpu.SMEM),
            pl.BlockSpec((5, 4, BR, QW), lambda i: (0, 0, i, 0)),
        ],
        out_specs=pl.BlockSpec((6, BR, N), lambda i: (0, i, 0)),
        out_shape=jax.ShapeDtypeStruct((6, N, N), jnp.bfloat16),
        compiler_params=pltpu.CompilerParams(
            dimension_semantics=("arbitrary",)),
    )(F, A)


def _mm_plain_body(a_ref, b_ref, out_ref, cs_ref, acc_ref):
    i = pl.program_id(2)
    j = pl.program_id(1)
    k = pl.program_id(3)

    @pl.when(k == 0)
    def _():
        acc_ref[...] = jnp.zeros_like(acc_ref)

    acc_ref[...] += jnp.dot(a_ref[0], b_ref[0],
                            preferred_element_type=jnp.float32)

    @pl.when(k == KB - 1)
    def _():
        r = acc_ref[...]
        ir = jax.lax.broadcasted_iota(jnp.int32, (BM, BN), 0) + i * BM
        ic = jax.lax.broadcasted_iota(jnp.int32, (BM, BN), 1) + j * BN
        r = jnp.where(ir == ic, 0.0, r)
        out_ref[0] = r.astype(jnp.bfloat16)
        part = jnp.sum(r, axis=0, keepdims=True)

        @pl.when(i == 0)
        def _():
            cs_ref[0, 0] = part

        @pl.when(i != 0)
        def _():
            cs_ref[0, 0] += part


def _mm_scaled_body(csin_ref, a_ref, b_ref, out_ref, cs_ref, acc_ref):
    i = pl.program_id(2)
    j = pl.program_id(1)
    k = pl.program_id(3)

    @pl.when(k == 0)
    def _():
        acc_ref[...] = jnp.zeros_like(acc_ref)

    sc = csin_ref[0, 0]
    dinv = jnp.where(sc != 0, 1.0 / jnp.where(sc != 0, sc, 1.0), 0.0)
    acc_ref[...] += jnp.dot((a_ref[0] * dinv).astype(jnp.bfloat16),
                            b_ref[0], preferred_element_type=jnp.float32)

    @pl.when(k == KB - 1)
    def _():
        r = acc_ref[...]
        ir = jax.lax.broadcasted_iota(jnp.int32, (BM, BN), 0) + i * BM
        ic = jax.lax.broadcasted_iota(jnp.int32, (BM, BN), 1) + j * BN
        r = jnp.where(ir == ic, 0.0, r)
        out_ref[0] = r.astype(jnp.bfloat16)
        part = jnp.sum(r, axis=0, keepdims=True)

        @pl.when(i == 0)
        def _():
            cs_ref[0, 0] = part

        @pl.when(i != 0)
        def _():
            cs_ref[0, 0] += part


def _mm(a, b, cs=None):
    grid = (2, N // BN, N // BM, KB)
    specs = [
        pl.BlockSpec((1, BM, BK), lambda c, j, i, k: (c, i, k)),
        pl.BlockSpec((1, BK, BN), lambda c, j, i, k: (c, k, j)),
    ]
    args = [a, b]
    body = _mm_plain_body
    if cs is not None:
        specs.insert(0, pl.BlockSpec((1, 1, 1, BK),
                                     lambda c, j, i, k: (c, k, 0, 0)))
        args.insert(0, cs.reshape(2, KB, 1, BK))
        body = _mm_scaled_body
    return pl.pallas_call(
        body,
        grid=grid,
        in_specs=specs,
        out_specs=[
            pl.BlockSpec((1, BM, BN), lambda c, j, i, k: (c, i, j)),
            pl.BlockSpec((1, 1, 1, BN), lambda c, j, i, k: (c, j, 0, 0)),
        ],
        out_shape=[
            jax.ShapeDtypeStruct((2, N, N), jnp.bfloat16),
            jax.ShapeDtypeStruct((2, N // BN, 1, BN), jnp.float32),
        ],
        scratch_shapes=[pltpu.VMEM((BM, BN), jnp.float32)],
        compiler_params=pltpu.CompilerParams(
            dimension_semantics=("parallel", "parallel", "arbitrary",
                                 "arbitrary")),
    )(*args)


def _xw_body(x_ref, w_ref, out_ref):
    out_ref[...] = jnp.dot(x_ref[...], w_ref[...],
                           preferred_element_type=jnp.float32)


def _xw(X, gcn_w):
    return pl.pallas_call(
        _xw_body,
        out_shape=jax.ShapeDtypeStruct((N, 128), jnp.float32),
    )(X, gcn_w)


def _final_body(h1_ref, xwTf_ref, xwTb_ref, csf_ref, csb_ref, gb_ref,
                w1t_ref, b1_ref, w2t_ref, b2_ref, out_ref):
    outs = []
    for c in range(2):
        csf = csf_ref[c:c + 1, :]
        dinv_f = jax.lax.rsqrt(1.0 + jnp.where(csf != 0, 1.0, 0.0))
        Yt = (xwTf_ref[...] * dinv_f).astype(jnp.bfloat16)
        Zt = jnp.dot(Yt, h1_ref[c], preferred_element_type=jnp.float32)
        csb = csb_ref[c:c + 1, :]
        nz = jnp.where(csb != 0, 1.0, 0.0)
        dinv1 = jnp.where(csb != 0,
                          1.0 / jnp.where(csb != 0, csb, 1.0), 0.0)
        dinv_b = jax.lax.rsqrt(1.0 + nz)
        o = (Zt * (dinv_b * dinv1) + xwTb_ref[...] * (dinv_b * dinv_b)
             + gb_ref[...])
        outs.append(jnp.maximum(o, 0.0))
    xcat = jnp.concatenate(outs, axis=0)
    h = jnp.dot(w1t_ref[...], xcat, preferred_element_type=jnp.float32)
    h = jnp.maximum(h + b1_ref[...], 0.0)
    out_ref[...] = (jnp.dot(w2t_ref[...], h,
                            preferred_element_type=jnp.float32)
                    + b2_ref[...])


def _final(h1, xwT, cs1, gb, w1t, b1, w2t, b2):
    return pl.pallas_call(
        _final_body,
        grid=(N // BI,),
        in_specs=[
            pl.BlockSpec((2, N, BI), lambda i: (0, 0, i)),
            pl.BlockSpec((128, N), lambda i: (0, 0)),
            pl.BlockSpec((128, BI), lambda i: (0, i)),
            pl.BlockSpec((2, N), lambda i: (0, 0)),
            pl.BlockSpec((2, BI), lambda i: (0, i)),
            pl.BlockSpec((128, 1), lambda i: (0, 0)),
            pl.BlockSpec((128, 256), lambda i: (0, 0)),
            pl.BlockSpec((128, 1), lambda i: (0, 0)),
            pl.BlockSpec((128, 128), lambda i: (0, 0)),
            pl.BlockSpec((128, 1), lambda i: (0, 0)),
        ],
        out_specs=pl.BlockSpec((128, BI), lambda i: (0, i)),
        out_shape=jax.ShapeDtypeStruct((128, N), jnp.float32),
        compiler_params=pltpu.CompilerParams(
            dimension_semantics=("arbitrary",)),
    )(h1, xwT, xwT, cs1, cs1, gb, w1t, b1, w2t, b2)


def kernel(edge_indices, edge_values, X, conv_w1_0, conv_w2_0, conv_w1_1,
           gcn_w, gcn_b, lin1_w, lin1_b, lin2_w, lin2_b):
    F = jnp.concatenate([
        jax.nn.softmax(conv_w1_0, axis=1),
        jax.nn.softmax(conv_w2_0, axis=1),
        jax.nn.softmax(conv_w1_1, axis=1),
    ], axis=0)
    A = _build_dense(edge_indices, edge_values)
    P = _combos(F, A)
    H0, cs0 = _mm(P[0:2], P[2:4])
    H1, cs1 = _mm(H0, P[4:6], cs=cs0)
    cs1 = cs1.reshape(2, N)
    xwT = _xw(X, gcn_w).T
    yT = _final(H1, xwT, cs1, gcn_b.reshape(128, 1),
                lin1_w.T, lin1_b.reshape(128, 1),
                lin2_w.T, lin2_b.reshape(128, 1))
    return yT.T


# speedup vs baseline: 3.0972x; 1.0767x over previous
import functools

import jax
import jax.numpy as jnp
from jax import lax
from jax.experimental import pallas as pl
from jax.experimental.pallas import tpu as pltpu
from jax.experimental.pallas import tpu_sc as plsc

N = 2048
BM = BN = 1024
BK = 1024
KB = N // BK
BI = 512
BR = 64


QW = 512
QWORDS = N * QW
SINK = QWORDS
_EDGES_PER_TILE = 65536 // 16


def _sc_body(ei_hbm, ev_hbm, out_hbm, rows_v, cols_v, vals_v, idx_v,
             zero_v, acc_sh, dsem):
    cid = lax.axis_index("c")
    sid = lax.axis_index("s")
    ept = _EDGES_PER_TILE

    def zinit(i, carry):
        zero_v[pl.ds(i * 16, 16)] = jnp.zeros((16,), jnp.float32)
        return carry
    lax.fori_loop(0, 1024, zinit, 0)

    lane8 = lax.iota(jnp.int32, 16) * 8

    def slice_body(s, carry):
        slice_id = cid * 10 + s
        e = slice_id // 4
        q = slice_id % 4
        base = q * QW

        pltpu.sync_copy(ei_hbm.at[e, 0, pl.ds(sid * ept, ept)], rows_v)
        pltpu.sync_copy(ei_hbm.at[e, 1, pl.ds(sid * ept, ept)], cols_v)
        pltpu.sync_copy(ev_hbm.at[e, pl.ds(sid * ept, ept)], vals_v)

        def body(i, c2):
            r = rows_v[pl.ds(i * 16, 16)]
            c = cols_v[pl.ds(i * 16, 16)]
            m = (c >= base) & (c < base + QW)
            flat = r * QW + (c - base)
            idx_v[pl.ds(i * 16, 16)] = jnp.where(m, flat, SINK + lane8)
            return c2
        lax.fori_loop(0, ept // 16, body, 0)

        @pl.when(s > 0)
        def _():
            pltpu.make_async_copy(
                acc_sh.at[pl.ds(sid * 65536, 65536)],
                out_hbm.at[0, 0, 0], dsem).wait()

        for z in range(4):
            pltpu.sync_copy(
                zero_v, acc_sh.at[pl.ds(sid * 65536 + z * 16384, 16384)])
        plsc.subcore_barrier()

        pltpu.sync_copy(vals_v, acc_sh.at[idx_v], add=True)
        plsc.subcore_barrier()

        pltpu.async_copy(acc_sh.at[pl.ds(sid * 65536, 65536)],
                         out_hbm.at[e, q, sid], dsem)
        return carry

    lax.fori_loop(0, 10, slice_body, 0)
    pltpu.make_async_copy(acc_sh.at[pl.ds(sid * 65536, 65536)],
                          out_hbm.at[0, 0, 0], dsem).wait()


def _build_dense(edge_indices, edge_values):
    mesh = plsc.VectorSubcoreMesh(core_axis_name="c", subcore_axis_name="s")
    k = functools.partial(
        pl.kernel,
        out_type=jax.ShapeDtypeStruct((5, 4, 16, 65536), jnp.float32),
        mesh=mesh,
        scratch_types=[
            pltpu.VMEM((_EDGES_PER_TILE,), jnp.int32),
            pltpu.VMEM((_EDGES_PER_TILE,), jnp.int32),
            pltpu.VMEM((_EDGES_PER_TILE,), jnp.float32),
            pltpu.VMEM((_EDGES_PER_TILE,), jnp.int32),
            pltpu.VMEM((16384,), jnp.float32),
            pltpu.VMEM_SHARED((QWORDS + 128,), jnp.float32),
            pltpu.SemaphoreType.DMA,
        ],
    )(_sc_body)
    out = k(edge_indices, edge_values)
    return out.reshape(5, 4, N, QW)


def _combo_body(f_ref, a_ref, out_ref):
    for c in range(6):
        for q in range(4):
            acc = f_ref[c, 0] * a_ref[0, q]
            for e in range(1, 5):
                acc = acc + f_ref[c, e] * a_ref[e, q]
            out_ref[c, :, q * QW:(q + 1) * QW] = acc.astype(jnp.bfloat16)


def _combos(F, A):
    return pl.pallas_call(
        _combo_body,
        grid=(N // BR,),
        in_specs=[
            pl.BlockSpec(memory_space=pltpu.SMEM),
            pl.BlockSpec((5, 4, BR, QW), lambda i: (0, 0, i, 0)),
        ],
        out_specs=pl.BlockSpec((6, BR, N), lambda i: (0, i, 0)),
        out_shape=jax.ShapeDtypeStruct((6, N, N), jnp.bfloat16),
        compiler_params=pltpu.CompilerParams(
            dimension_semantics=("arbitrary",)),
    )(F, A)


def _mm_plain_body(a_ref, b_ref, out_ref, cs_ref, acc_ref):
    i = pl.program_id(2)
    j = pl.program_id(1)
    k = pl.program_id(3)

    @pl.when(k == 0)
    def _():
        acc_ref[...] = jnp.zeros_like(acc_ref)

    acc_ref[...] += jnp.dot(a_ref[0], b_ref[0],
                            preferred_element_type=jnp.float32)

    @pl.when(k == KB - 1)
    def _():
        r = acc_ref[...]
        ir = jax.lax.broadcasted_iota(jnp.int32, (BM, BN), 0) + i * BM
        ic = jax.lax.broadcasted_iota(jnp.int32, (BM, BN), 1) + j * BN
        r = jnp.where(ir == ic, 0.0, r)
        out_ref[0] = r.astype(jnp.bfloat16)
        part = jnp.sum(r, axis=0, keepdims=True)

        @pl.when(i == 0)
        def _():
            cs_ref[0, 0] = part

        @pl.when(i != 0)
        def _():
            cs_ref[0, 0] += part


def _mm_scaled_body(csin_ref, a_ref, b_ref, out_ref, cs_ref, acc_ref):
    i = pl.program_id(2)
    j = pl.program_id(1)
    k = pl.program_id(3)

    @pl.when(k == 0)
    def _():
        acc_ref[...] = jnp.zeros_like(acc_ref)

    sc = csin_ref[0, 0]
    dinv = jnp.where(sc != 0, 1.0 / jnp.where(sc != 0, sc, 1.0), 0.0)
    acc_ref[...] += jnp.dot((a_ref[0] * dinv).astype(jnp.bfloat16),
                            b_ref[0], preferred_element_type=jnp.float32)

    @pl.when(k == KB - 1)
    def _():
        r = acc_ref[...]
        ir = jax.lax.broadcasted_iota(jnp.int32, (BM, BN), 0) + i * BM
        ic = jax.lax.broadcasted_iota(jnp.int32, (BM, BN), 1) + j * BN
        r = jnp.where(ir == ic, 0.0, r)
        out_ref[0] = r.astype(jnp.bfloat16)
        part = jnp.sum(r, axis=0, keepdims=True)

        @pl.when(i == 0)
        def _():
            cs_ref[0, 0] = part

        @pl.when(i != 0)
        def _():
            cs_ref[0, 0] += part


def _mm(a, b, cs=None):
    grid = (2, N // BN, N // BM, KB)
    specs = [
        pl.BlockSpec((1, BM, BK), lambda c, j, i, k: (c, i, k)),
        pl.BlockSpec((1, BK, BN), lambda c, j, i, k: (c, k, j)),
    ]
    args = [a, b]
    body = _mm_plain_body
    if cs is not None:
        specs.insert(0, pl.BlockSpec((1, 1, 1, BK),
                                     lambda c, j, i, k: (c, k, 0, 0)))
        args.insert(0, cs.reshape(2, KB, 1, BK))
        body = _mm_scaled_body
    return pl.pallas_call(
        body,
        grid=grid,
        in_specs=specs,
        out_specs=[
            pl.BlockSpec((1, BM, BN), lambda c, j, i, k: (c, i, j)),
            pl.BlockSpec((1, 1, 1, BN), lambda c, j, i, k: (c, j, 0, 0)),
        ],
        out_shape=[
            jax.ShapeDtypeStruct((2, N, N), jnp.bfloat16),
            jax.ShapeDtypeStruct((2, N // BN, 1, BN), jnp.float32),
        ],
        scratch_shapes=[pltpu.VMEM((BM, BN), jnp.float32)],
        compiler_params=pltpu.CompilerParams(
            dimension_semantics=("parallel", "parallel", "arbitrary",
                                 "arbitrary")),
    )(*args)


def _xw_body(x_ref, w_ref, out_ref):
    out_ref[...] = jnp.dot(x_ref[...], w_ref[...],
                           preferred_element_type=jnp.float32)


def _xw(X, gcn_w):
    return pl.pallas_call(
        _xw_body,
        out_shape=jax.ShapeDtypeStruct((N, 128), jnp.float32),
    )(X, gcn_w)


def _final_body(h1_ref, xwTf_ref, xwTb_ref, csf_ref, csb_ref, gb_ref,
                w1t_ref, b1_ref, w2t_ref, b2_ref, out_ref):
    outs = []
    for c in range(2):
        csf = csf_ref[c:c + 1, :]
        dinv_f = jax.lax.rsqrt(1.0 + jnp.where(csf != 0, 1.0, 0.0))
        Yt = (xwTf_ref[...] * dinv_f).astype(jnp.bfloat16)
        Zt = jnp.dot(Yt, h1_ref[c], preferred_element_type=jnp.float32)
        csb = csb_ref[c:c + 1, :]
        nz = jnp.where(csb != 0, 1.0, 0.0)
        dinv1 = jnp.where(csb != 0,
                          1.0 / jnp.where(csb != 0, csb, 1.0), 0.0)
        dinv_b = jax.lax.rsqrt(1.0 + nz)
        o = (Zt * (dinv_b * dinv1) + xwTb_ref[...] * (dinv_b * dinv_b)
             + gb_ref[...])
        outs.append(jnp.maximum(o, 0.0))
    xcat = jnp.concatenate(outs, axis=0)
    h = jnp.dot(w1t_ref[...], xcat, preferred_element_type=jnp.float32)
    h = jnp.maximum(h + b1_ref[...], 0.0)
    out_ref[...] = (jnp.dot(w2t_ref[...], h,
                            preferred_element_type=jnp.float32)
                    + b2_ref[...])


def _final(h1, xwT, cs1, gb, w1t, b1, w2t, b2):
    return pl.pallas_call(
        _final_body,
        grid=(N // BI,),
        in_specs=[
            pl.BlockSpec((2, N, BI), lambda i: (0, 0, i)),
            pl.BlockSpec((128, N), lambda i: (0, 0)),
            pl.BlockSpec((128, BI), lambda i: (0, i)),
            pl.BlockSpec((2, N), lambda i: (0, 0)),
            pl.BlockSpec((2, BI), lambda i: (0, i)),
            pl.BlockSpec((128, 1), lambda i: (0, 0)),
            pl.BlockSpec((128, 256), lambda i: (0, 0)),
            pl.BlockSpec((128, 1), lambda i: (0, 0)),
            pl.BlockSpec((128, 128), lambda i: (0, 0)),
            pl.BlockSpec((128, 1), lambda i: (0, 0)),
        ],
        out_specs=pl.BlockSpec((128, BI), lambda i: (0, i)),
        out_shape=jax.ShapeDtypeStruct((128, N), jnp.float32),
        compiler_params=pltpu.CompilerParams(
            dimension_semantics=("arbitrary",)),
    )(h1, xwT, xwT, cs1, cs1, gb, w1t, b1, w2t, b2)


def kernel(edge_indices, edge_values, X, conv_w1_0, conv_w2_0, conv_w1_1,
           gcn_w, gcn_b, lin1_w, lin1_b, lin2_w, lin2_b):
    F = jnp.concatenate([
        jax.nn.softmax(conv_w1_0, axis=1),
        jax.nn.softmax(conv_w2_0, axis=1),
        jax.nn.softmax(conv_w1_1, axis=1),
    ], axis=0)
    A = _build_dense(edge_indices, edge_values)
    P = _combos(F, A)
    H0, cs0 = _mm(P[0:2], P[2:4])
    H1, cs1 = _mm(H0, P[4:6], cs=cs0)
    cs1 = cs1.reshape(2, N)
    xwT = _xw(X, gcn_w).T
    yT = _final(H1, xwT, cs1, gcn_b.reshape(128, 1),
                lin1_w.T, lin1_b.reshape(128, 1),
                lin2_w.T, lin2_b.reshape(128, 1))
    return yT.T


# speedup vs baseline: 3.1101x; 1.0042x over previous
import functools

import jax
import jax.numpy as jnp
from jax import lax
from jax.experimental import pallas as pl
from jax.experimental.pallas import tpu as pltpu
from jax.experimental.pallas import tpu_sc as plsc

N = 2048
BM = BN = 1024
BK = 1024
KB = N // BK
BI = 512
BR = 64


QW = 512
QWORDS = N * QW
SINK = QWORDS
_EDGES_PER_TILE = 65536 // 16


def _sc_body(ei_hbm, ev_hbm, out_hbm, rows_v, cols_v, vals_v, idx_v,
             zero_v, acc_sh, dsem, zsem):
    cid = lax.axis_index("c")
    sid = lax.axis_index("s")
    ept = _EDGES_PER_TILE

    def zinit(i, carry):
        zero_v[pl.ds(i * 16, 16)] = jnp.zeros((16,), jnp.float32)
        return carry
    lax.fori_loop(0, 1024, zinit, 0)

    lane8 = lax.iota(jnp.int32, 16) * 8

    def slice_body(s, carry):
        slice_id = cid * 10 + s
        e = slice_id // 4
        q = slice_id % 4
        base = q * QW

        pltpu.sync_copy(ei_hbm.at[e, 0, pl.ds(sid * ept, ept)], rows_v)
        pltpu.sync_copy(ei_hbm.at[e, 1, pl.ds(sid * ept, ept)], cols_v)
        pltpu.sync_copy(ev_hbm.at[e, pl.ds(sid * ept, ept)], vals_v)

        def body(i, c2):
            r = rows_v[pl.ds(i * 16, 16)]
            c = cols_v[pl.ds(i * 16, 16)]
            m = (c >= base) & (c < base + QW)
            flat = r * QW + (c - base)
            idx_v[pl.ds(i * 16, 16)] = jnp.where(m, flat, SINK + lane8)
            return c2
        lax.fori_loop(0, ept // 16, body, 0)

        @pl.when(s > 0)
        def _():
            pltpu.make_async_copy(
                acc_sh.at[pl.ds(sid * 65536, 65536)],
                out_hbm.at[0, 0, 0], dsem).wait()

        zcopies = [
            pltpu.async_copy(
                zero_v, acc_sh.at[pl.ds(sid * 65536 + z * 16384, 16384)],
                zsem)
            for z in range(4)
        ]
        for zc in zcopies:
            zc.wait()
        plsc.subcore_barrier()

        pltpu.sync_copy(vals_v, acc_sh.at[idx_v], add=True)
        plsc.subcore_barrier()

        pltpu.async_copy(acc_sh.at[pl.ds(sid * 65536, 65536)],
                         out_hbm.at[e, q, sid], dsem)
        return carry

    lax.fori_loop(0, 10, slice_body, 0)
    pltpu.make_async_copy(acc_sh.at[pl.ds(sid * 65536, 65536)],
                          out_hbm.at[0, 0, 0], dsem).wait()


def _build_dense(edge_indices, edge_values):
    mesh = plsc.VectorSubcoreMesh(core_axis_name="c", subcore_axis_name="s")
    k = functools.partial(
        pl.kernel,
        out_type=jax.ShapeDtypeStruct((5, 4, 16, 65536), jnp.float32),
        mesh=mesh,
        scratch_types=[
            pltpu.VMEM((_EDGES_PER_TILE,), jnp.int32),
            pltpu.VMEM((_EDGES_PER_TILE,), jnp.int32),
            pltpu.VMEM((_EDGES_PER_TILE,), jnp.float32),
            pltpu.VMEM((_EDGES_PER_TILE,), jnp.int32),
            pltpu.VMEM((16384,), jnp.float32),
            pltpu.VMEM_SHARED((QWORDS + 128,), jnp.float32),
            pltpu.SemaphoreType.DMA,
            pltpu.SemaphoreType.DMA,
        ],
    )(_sc_body)
    out = k(edge_indices, edge_values)
    return out.reshape(5, 4, N, QW)


def _combo_body(f_ref, a_ref, out_ref):
    for c in range(6):
        for q in range(4):
            acc = f_ref[c, 0] * a_ref[0, q]
            for e in range(1, 5):
                acc = acc + f_ref[c, e] * a_ref[e, q]
            out_ref[c, :, q * QW:(q + 1) * QW] = acc.astype(jnp.bfloat16)


def _combos(F, A):
    return pl.pallas_call(
        _combo_body,
        grid=(N // BR,),
        in_specs=[
            pl.BlockSpec(memory_space=pltpu.SMEM),
            pl.BlockSpec((5, 4, BR, QW), lambda i: (0, 0, i, 0)),
        ],
        out_specs=pl.BlockSpec((6, BR, N), lambda i: (0, i, 0)),
        out_shape=jax.ShapeDtypeStruct((6, N, N), jnp.bfloat16),
        compiler_params=pltpu.CompilerParams(
            dimension_semantics=("arbitrary",)),
    )(F, A)


def _mm_plain_body(a_ref, b_ref, out_ref, cs_ref, acc_ref):
    i = pl.program_id(2)
    j = pl.program_id(1)
    k = pl.program_id(3)

    @pl.when(k == 0)
    def _():
        acc_ref[...] = jnp.zeros_like(acc_ref)

    acc_ref[...] += jnp.dot(a_ref[0], b_ref[0],
                            preferred_element_type=jnp.float32)

    @pl.when(k == KB - 1)
    def _():
        r = acc_ref[...]
        ir = jax.lax.broadcasted_iota(jnp.int32, (BM, BN), 0) + i * BM
        ic = jax.lax.broadcasted_iota(jnp.int32, (BM, BN), 1) + j * BN
        r = jnp.where(ir == ic, 0.0, r)
        out_ref[0] = r.astype(jnp.bfloat16)
        part = jnp.sum(r, axis=0, keepdims=True)

        @pl.when(i == 0)
        def _():
            cs_ref[0, 0] = part

        @pl.when(i != 0)
        def _():
            cs_ref[0, 0] += part


def _mm_scaled_body(csin_ref, a_ref, b_ref, out_ref, cs_ref, acc_ref):
    i = pl.program_id(2)
    j = pl.program_id(1)
    k = pl.program_id(3)

    @pl.when(k == 0)
    def _():
        acc_ref[...] = jnp.zeros_like(acc_ref)

    sc = csin_ref[0, 0]
    dinv = jnp.where(sc != 0, 1.0 / jnp.where(sc != 0, sc, 1.0), 0.0)
    acc_ref[...] += jnp.dot((a_ref[0] * dinv).astype(jnp.bfloat16),
                            b_ref[0], preferred_element_type=jnp.float32)

    @pl.when(k == KB - 1)
    def _():
        r = acc_ref[...]
        ir = jax.lax.broadcasted_iota(jnp.int32, (BM, BN), 0) + i * BM
        ic = jax.lax.broadcasted_iota(jnp.int32, (BM, BN), 1) + j * BN
        r = jnp.where(ir == ic, 0.0, r)
        out_ref[0] = r.astype(jnp.bfloat16)
        part = jnp.sum(r, axis=0, keepdims=True)

        @pl.when(i == 0)
        def _():
            cs_ref[0, 0] = part

        @pl.when(i != 0)
        def _():
            cs_ref[0, 0] += part


def _mm(a, b, cs=None):
    grid = (2, N // BN, N // BM, KB)
    specs = [
        pl.BlockSpec((1, BM, BK), lambda c, j, i, k: (c, i, k)),
        pl.BlockSpec((1, BK, BN), lambda c, j, i, k: (c, k, j)),
    ]
    args = [a, b]
    body = _mm_plain_body
    if cs is not None:
        specs.insert(0, pl.BlockSpec((1, 1, 1, BK),
                                     lambda c, j, i, k: (c, k, 0, 0)))
        args.insert(0, cs.reshape(2, KB, 1, BK))
        body = _mm_scaled_body
    return pl.pallas_call(
        body,
        grid=grid,
        in_specs=specs,
        out_specs=[
            pl.BlockSpec((1, BM, BN), lambda c, j, i, k: (c, i, j)),
            pl.BlockSpec((1, 1, 1, BN), lambda c, j, i, k: (c, j, 0, 0)),
        ],
        out_shape=[
            jax.ShapeDtypeStruct((2, N, N), jnp.bfloat16),
            jax.ShapeDtypeStruct((2, N // BN, 1, BN), jnp.float32),
        ],
        scratch_shapes=[pltpu.VMEM((BM, BN), jnp.float32)],
        compiler_params=pltpu.CompilerParams(
            dimension_semantics=("parallel", "parallel", "arbitrary",
                                 "arbitrary")),
    )(*args)


def _xw_body(x_ref, w_ref, out_ref):
    out_ref[...] = jnp.dot(x_ref[...], w_ref[...],
                           preferred_element_type=jnp.float32)


def _xw(X, gcn_w):
    return pl.pallas_call(
        _xw_body,
        out_shape=jax.ShapeDtypeStruct((N, 128), jnp.float32),
    )(X, gcn_w)


def _final_body(h1_ref, xwTf_ref, xwTb_ref, csf_ref, csb_ref, gb_ref,
                w1t_ref, b1_ref, w2t_ref, b2_ref, out_ref):
    outs = []
    for c in range(2):
        csf = csf_ref[c:c + 1, :]
        dinv_f = jax.lax.rsqrt(1.0 + jnp.where(csf != 0, 1.0, 0.0))
        Yt = (xwTf_ref[...] * dinv_f).astype(jnp.bfloat16)
        Zt = jnp.dot(Yt, h1_ref[c], preferred_element_type=jnp.float32)
        csb = csb_ref[c:c + 1, :]
        nz = jnp.where(csb != 0, 1.0, 0.0)
        dinv1 = jnp.where(csb != 0,
                          1.0 / jnp.where(csb != 0, csb, 1.0), 0.0)
        dinv_b = jax.lax.rsqrt(1.0 + nz)
        o = (Zt * (dinv_b * dinv1) + xwTb_ref[...] * (dinv_b * dinv_b)
             + gb_ref[...])
        outs.append(jnp.maximum(o, 0.0))
    xcat = jnp.concatenate(outs, axis=0)
    h = jnp.dot(w1t_ref[...], xcat, preferred_element_type=jnp.float32)
    h = jnp.maximum(h + b1_ref[...], 0.0)
    out_ref[...] = (jnp.dot(w2t_ref[...], h,
                            preferred_element_type=jnp.float32)
                    + b2_ref[...])


def _final(h1, xwT, cs1, gb, w1t, b1, w2t, b2):
    return pl.pallas_call(
        _final_body,
        grid=(N // BI,),
        in_specs=[
            pl.BlockSpec((2, N, BI), lambda i: (0, 0, i)),
            pl.BlockSpec((128, N), lambda i: (0, 0)),
            pl.BlockSpec((128, BI), lambda i: (0, i)),
            pl.BlockSpec((2, N), lambda i: (0, 0)),
            pl.BlockSpec((2, BI), lambda i: (0, i)),
            pl.BlockSpec((128, 1), lambda i: (0, 0)),
            pl.BlockSpec((128, 256), lambda i: (0, 0)),
            pl.BlockSpec((128, 1), lambda i: (0, 0)),
            pl.BlockSpec((128, 128), lambda i: (0, 0)),
            pl.BlockSpec((128, 1), lambda i: (0, 0)),
        ],
        out_specs=pl.BlockSpec((128, BI), lambda i: (0, i)),
        out_shape=jax.ShapeDtypeStruct((128, N), jnp.float32),
        compiler_params=pltpu.CompilerParams(
            dimension_semantics=("arbitrary",)),
    )(h1, xwT, xwT, cs1, cs1, gb, w1t, b1, w2t, b2)


def kernel(edge_indices, edge_values, X, conv_w1_0, conv_w2_0, conv_w1_1,
           gcn_w, gcn_b, lin1_w, lin1_b, lin2_w, lin2_b):
    F = jnp.concatenate([
        jax.nn.softmax(conv_w1_0, axis=1),
        jax.nn.softmax(conv_w2_0, axis=1),
        jax.nn.softmax(conv_w1_1, axis=1),
    ], axis=0)
    A = _build_dense(edge_indices, edge_values)
    P = _combos(F, A)
    H0, cs0 = _mm(P[0:2], P[2:4])
    H1, cs1 = _mm(H0, P[4:6], cs=cs0)
    cs1 = cs1.reshape(2, N)
    xwT = _xw(X, gcn_w).T
    yT = _final(H1, xwT, cs1, gcn_b.reshape(128, 1),
                lin1_w.T, lin1_b.reshape(128, 1),
                lin2_w.T, lin2_b.reshape(128, 1))
    return yT.T
